# baseline (speedup 1.0000x reference)
import functools

import jax
import jax.numpy as jnp
from jax import lax
from jax.experimental import pallas as pl
from jax.experimental.pallas import tpu as pltpu
from jax.experimental.pallas import tpu_sc as plsc

PROTO_NUM = 4
N_CLASSES = 1000
FEAT_DIM = 512
BATCH = 16384

L = 16
NC = 2
NS = 16
NW = NC * NS
PER_W = BATCH // NW
CHUNK = 32
NCHUNK = PER_W // CHUNK
GROUP = BATCH // PROTO_NUM
VPR = FEAT_DIM // L


def _rsqrt(x):
    i = lax.bitcast_convert_type(x, jnp.int32)
    i = jnp.int32(0x5F3759DF) - lax.shift_right_arithmetic(i, 1)
    y = lax.bitcast_convert_type(i, jnp.float32)
    for _ in range(3):
        y = y * (jnp.float32(1.5) - jnp.float32(0.5) * x * y * y)
    return y


def _sqrt(x):
    return x * _rsqrt(jnp.maximum(x, jnp.float32(1e-35)))


def _body(feat_hbm, y_hbm, table_hbm, out_hbm,
          idx_v, f0, f1, k0, k1, loss_v, sf0, sf1, sk0, sk1):
    cid = lax.axis_index("c")
    sid = lax.axis_index("s")
    wid = sid * NC + cid
    base = wid * PER_W
    goff = (base // GROUP) * N_CLASSES

    pltpu.sync_copy(y_hbm.at[pl.ds(base, PER_W)], idx_v)
    for j in range(PER_W // L):
        sl = pl.ds(j * L, L)
        idx_v[sl] = idx_v[sl] + goff

    fbufs = (f0, f1)
    kbufs = (k0, k1)
    fsems = (sf0, sf1)
    ksems = (sk0, sk1)

    def fdesc(b, c):
        return pltpu.make_async_copy(
            feat_hbm.at[pl.ds(base + c * CHUNK, CHUNK)], fbufs[b], fsems[b])

    def kdesc(b, c):
        return pltpu.make_async_copy(
            table_hbm.at[idx_v.at[pl.ds(c * CHUNK, CHUNK)]], kbufs[b], ksems[b])

    fdesc(0, 0).start()
    kdesc(0, 0).start()
    fdesc(1, 1).start()
    kdesc(1, 1).start()

    def chunk_pair(g, acc):
        for b in range(2):
            c = 2 * g + b
            fdesc(b, c).wait()
            kdesc(b, c).wait()
            acc = compute_chunk(fbufs[b], kbufs[b], acc)
            c2 = jnp.minimum(c + 2, NCHUNK - 1)
            fdesc(b, c2).start()
            kdesc(b, c2).start()
        return acc

    def compute_chunk(fb, kb, acc):

        def dist(s):
            ff = jnp.zeros((L,), jnp.float32)
            kk = jnp.zeros((L,), jnp.float32)
            fk = jnp.zeros((L,), jnp.float32)
            for j in range(VPR):
                sl = pl.ds(j * L, L)
                fv = fb[s, sl]
                kv = kb[s, sl]
                ff = ff + fv * fv
                kk = kk + kv * kv
                fk = fk + fv * kv
            ffs = jnp.sum(ff)
            kks = jnp.sum(kk)
            fks = jnp.sum(fk)
            inv = _rsqrt(jnp.maximum(
                jnp.broadcast_to(ffs * kks, (L,)), jnp.float32(1e-35)))
            cos = jnp.broadcast_to(fks, (L,)) * inv
            d2 = jnp.maximum(jnp.float32(2.0) - jnp.float32(2.0) * cos,
                             jnp.float32(0.0))
            return _sqrt(d2)

        def sample2(i, a):
            return a + dist(2 * i) + dist(2 * i + 1)

        return lax.fori_loop(0, CHUNK // 2, sample2, acc)

    acc = jnp.zeros((L,), jnp.float32)
    acc = lax.fori_loop(0, NCHUNK // 2, chunk_pair, acc)

    for b in range(2):
        fdesc(b, NCHUNK - 1).wait()
        kdesc(b, NCHUNK - 1).wait()

    loss_v[...] = acc
    pltpu.sync_copy(loss_v, out_hbm.at[wid])


@jax.jit
def kernel(feature, y, y_pred, prototype):
    del y_pred
    table = jnp.reshape(prototype, (PROTO_NUM * N_CLASSES, FEAT_DIM))
    mesh = plsc.VectorSubcoreMesh(core_axis_name="c", subcore_axis_name="s")
    partial = pl.kernel(
        _body,
        out_type=jax.ShapeDtypeStruct((NW, L), jnp.float32),
        mesh=mesh,
        compiler_params=pltpu.CompilerParams(needs_layout_passes=False),
        scratch_types=[
            pltpu.VMEM((PER_W,), jnp.int32),
            pltpu.VMEM((CHUNK, FEAT_DIM), jnp.float32),
            pltpu.VMEM((CHUNK, FEAT_DIM), jnp.float32),
            pltpu.VMEM((CHUNK, FEAT_DIM), jnp.float32),
            pltpu.VMEM((CHUNK, FEAT_DIM), jnp.float32),
            pltpu.VMEM((L,), jnp.float32),
            pltpu.SemaphoreType.DMA,
            pltpu.SemaphoreType.DMA,
            pltpu.SemaphoreType.DMA,
            pltpu.SemaphoreType.DMA,
        ],
    )(feature, y, table)
    return jnp.sum(partial) / jnp.float32(L * BATCH)


# speedup vs baseline: 1.4372x; 1.4372x over previous
import functools

import jax
import jax.numpy as jnp
from jax import lax
from jax.experimental import pallas as pl
from jax.experimental.pallas import tpu as pltpu
from jax.experimental.pallas import tpu_sc as plsc

PROTO_NUM = 4
N_CLASSES = 1000
FEAT_DIM = 512
BATCH = 16384

L = 16
NC = 2
NS = 16
NW = NC * NS
PER_W = BATCH // NW
CHUNK = 32
NCHUNK = PER_W // CHUNK
GROUP = BATCH // PROTO_NUM
VPR = FEAT_DIM // L


def _rsqrt(x):
    i = lax.bitcast_convert_type(x, jnp.int32)
    i = jnp.int32(0x5F3759DF) - lax.shift_right_arithmetic(i, 1)
    y = lax.bitcast_convert_type(i, jnp.float32)
    for _ in range(3):
        y = y * (jnp.float32(1.5) - jnp.float32(0.5) * x * y * y)
    return y


def _sqrt(x):
    return x * _rsqrt(jnp.maximum(x, jnp.float32(1e-35)))


def _body(feat_hbm, y_hbm, table_hbm, out_hbm,
          idx_v, f0, f1, k0, k1, loss_v, sf0, sf1, sk0, sk1):
    cid = lax.axis_index("c")
    sid = lax.axis_index("s")
    wid = sid * NC + cid
    base = wid * PER_W
    goff = (base // GROUP) * N_CLASSES

    pltpu.sync_copy(y_hbm.at[pl.ds(base, PER_W)], idx_v)
    for j in range(PER_W // L):
        sl = pl.ds(j * L, L)
        idx_v[sl] = idx_v[sl] + goff

    fbufs = (f0, f1)
    kbufs = (k0, k1)
    fsems = (sf0, sf1)
    ksems = (sk0, sk1)

    def issue(c):
        b = c % 2
        fcp = pltpu.async_copy(
            feat_hbm.at[pl.ds(base + c * CHUNK, CHUNK)], fbufs[b], fsems[b])
        kcp = pltpu.async_copy(
            table_hbm.at[idx_v.at[pl.ds(c * CHUNK, CHUNK)]], kbufs[b], ksems[b])
        return fcp, kcp

    def compute_chunk(fb, kb, acc):

        def dist(s):
            ff = jnp.zeros((L,), jnp.float32)
            kk = jnp.zeros((L,), jnp.float32)
            fk = jnp.zeros((L,), jnp.float32)
            for j in range(VPR):
                sl = pl.ds(j * L, L)
                fv = fb[s, sl]
                kv = kb[s, sl]
                ff = ff + fv * fv
                kk = kk + kv * kv
                fk = fk + fv * kv
            ffs = jnp.sum(ff)
            kks = jnp.sum(kk)
            fks = jnp.sum(fk)
            inv = _rsqrt(jnp.maximum(
                jnp.broadcast_to(ffs * kks, (L,)), jnp.float32(1e-35)))
            cos = jnp.broadcast_to(fks, (L,)) * inv
            d2 = jnp.maximum(jnp.float32(2.0) - jnp.float32(2.0) * cos,
                             jnp.float32(0.0))
            return _sqrt(d2)

        def sample(s, a):
            return a + dist(s)

        return lax.fori_loop(0, CHUNK, sample, acc)

    acc = jnp.zeros((L,), jnp.float32)
    pending = issue(0)
    for c in range(NCHUNK):
        fcp, kcp = pending
        fcp.wait()
        kcp.wait()
        if c + 1 < NCHUNK:
            pending = issue(c + 1)
        acc = compute_chunk(fbufs[c % 2], kbufs[c % 2], acc)

    loss_v[...] = acc
    pltpu.sync_copy(loss_v, out_hbm.at[wid])


@jax.jit
def kernel(feature, y, y_pred, prototype):
    del y_pred
    table = jnp.reshape(prototype, (PROTO_NUM * N_CLASSES, FEAT_DIM))
    mesh = plsc.VectorSubcoreMesh(core_axis_name="c", subcore_axis_name="s")
    partial = pl.kernel(
        _body,
        out_type=jax.ShapeDtypeStruct((NW, L), jnp.float32),
        mesh=mesh,
        compiler_params=pltpu.CompilerParams(needs_layout_passes=False),
        scratch_types=[
            pltpu.VMEM((PER_W,), jnp.int32),
            pltpu.VMEM((CHUNK, FEAT_DIM), jnp.float32),
            pltpu.VMEM((CHUNK, FEAT_DIM), jnp.float32),
            pltpu.VMEM((CHUNK, FEAT_DIM), jnp.float32),
            pltpu.VMEM((CHUNK, FEAT_DIM), jnp.float32),
            pltpu.VMEM((L,), jnp.float32),
            pltpu.SemaphoreType.DMA,
            pltpu.SemaphoreType.DMA,
            pltpu.SemaphoreType.DMA,
            pltpu.SemaphoreType.DMA,
        ],
    )(feature, y, table)
    return jnp.sum(partial) / jnp.float32(L * BATCH)


# speedup vs baseline: 1.5328x; 1.0665x over previous
import functools

import jax
import jax.numpy as jnp
from jax import lax
from jax.experimental import pallas as pl
from jax.experimental.pallas import tpu as pltpu
from jax.experimental.pallas import tpu_sc as plsc

PROTO_NUM = 4
N_CLASSES = 1000
FEAT_DIM = 512
BATCH = 16384

L = 16
NC = 2
NS = 16
NW = NC * NS
PER_W = BATCH // NW
CHUNK = 32
NCHUNK = PER_W // CHUNK
GROUP = BATCH // PROTO_NUM
VPR = FEAT_DIM // L


def _rsqrt(x):
    i = lax.bitcast_convert_type(x, jnp.int32)
    i = jnp.int32(0x5F3759DF) - lax.shift_right_arithmetic(i, 1)
    y = lax.bitcast_convert_type(i, jnp.float32)
    for _ in range(2):
        y = y * (jnp.float32(1.5) - jnp.float32(0.5) * x * y * y)
    return y


def _sqrt(x):
    return x * _rsqrt(jnp.maximum(x, jnp.float32(1e-35)))


def _body(feat_hbm, y_hbm, table_hbm, out_hbm,
          idx_v, f0, f1, k0, k1, loss_v, sf0, sf1, sk0, sk1):
    cid = lax.axis_index("c")
    sid = lax.axis_index("s")
    wid = sid * NC + cid
    base = wid * PER_W
    goff = (base // GROUP) * N_CLASSES

    pltpu.sync_copy(y_hbm.at[pl.ds(base, PER_W)], idx_v)
    for j in range(PER_W // L):
        sl = pl.ds(j * L, L)
        idx_v[sl] = idx_v[sl] + goff

    fbufs = (f0, f1)
    kbufs = (k0, k1)
    fsems = (sf0, sf1)
    ksems = (sk0, sk1)

    def issue(c):
        b = c % 2
        fcp = pltpu.async_copy(
            feat_hbm.at[pl.ds(base + c * CHUNK, CHUNK)], fbufs[b], fsems[b])
        kcp = pltpu.async_copy(
            table_hbm.at[idx_v.at[pl.ds(c * CHUNK, CHUNK)]], kbufs[b], ksems[b])
        return fcp, kcp

    lane = lax.iota(jnp.int32, L)
    perms = [lax.bitwise_xor(lane, jnp.int32(sh)) for sh in (8, 4, 2, 1)]

    dnums = lax.GatherDimensionNumbers(
        offset_dims=(), collapsed_slice_dims=(0,), start_index_map=(0,))

    def shuffle(x, p):
        return lax.gather(
            x, p[:, None], dnums, (1,),
            mode=lax.GatherScatterMode.PROMISE_IN_BOUNDS)

    def lanesum(x):
        for p in perms:
            x = x + shuffle(x, p)
        return x

    def compute_chunk(fb, kb, acc):

        def dist(s):
            ff = jnp.zeros((L,), jnp.float32)
            kk = jnp.zeros((L,), jnp.float32)
            fk = jnp.zeros((L,), jnp.float32)
            for j in range(VPR):
                sl = pl.ds(j * L, L)
                fv = fb[s, sl]
                kv = kb[s, sl]
                ff = ff + fv * fv
                kk = kk + kv * kv
                fk = fk + fv * kv
            ffs = lanesum(ff)
            kks = lanesum(kk)
            fks = lanesum(fk)
            inv = _rsqrt(jnp.maximum(ffs * kks, jnp.float32(1e-35)))
            cos = fks * inv
            d2 = jnp.maximum(jnp.float32(2.0) - jnp.float32(2.0) * cos,
                             jnp.float32(0.0))
            return _sqrt(d2)

        def sample(s, a):
            return a + dist(s)

        return lax.fori_loop(0, CHUNK, sample, acc)

    acc = jnp.zeros((L,), jnp.float32)
    pending = issue(0)
    for c in range(NCHUNK):
        fcp, kcp = pending
        fcp.wait()
        kcp.wait()
        if c + 1 < NCHUNK:
            pending = issue(c + 1)
        acc = compute_chunk(fbufs[c % 2], kbufs[c % 2], acc)

    loss_v[...] = acc
    pltpu.sync_copy(loss_v, out_hbm.at[wid])


@jax.jit
def kernel(feature, y, y_pred, prototype):
    del y_pred
    table = jnp.reshape(prototype, (PROTO_NUM * N_CLASSES, FEAT_DIM))
    mesh = plsc.VectorSubcoreMesh(core_axis_name="c", subcore_axis_name="s")
    partial = pl.kernel(
        _body,
        out_type=jax.ShapeDtypeStruct((NW, L), jnp.float32),
        mesh=mesh,
        compiler_params=pltpu.CompilerParams(needs_layout_passes=False),
        scratch_types=[
            pltpu.VMEM((PER_W,), jnp.int32),
            pltpu.VMEM((CHUNK, FEAT_DIM), jnp.float32),
            pltpu.VMEM((CHUNK, FEAT_DIM), jnp.float32),
            pltpu.VMEM((CHUNK, FEAT_DIM), jnp.float32),
            pltpu.VMEM((CHUNK, FEAT_DIM), jnp.float32),
            pltpu.VMEM((L,), jnp.float32),
            pltpu.SemaphoreType.DMA,
            pltpu.SemaphoreType.DMA,
            pltpu.SemaphoreType.DMA,
            pltpu.SemaphoreType.DMA,
        ],
    )(feature, y, table)
    return jnp.sum(partial) / jnp.float32(L * BATCH)
